# Initial kernel scaffold; baseline (speedup 1.0000x reference)
import jax
import jax.numpy as jnp
from jax.experimental import pallas as pl


def kernel(indices, weights, bias):
    raise NotImplementedError("write your pallas kernel here")


# speedup vs baseline: 1.2922x; 1.2922x over previous
import functools

import jax
import jax.numpy as jnp
from jax import lax
from jax.experimental import pallas as pl
from jax.experimental.pallas import tpu as pltpu
from jax.experimental.pallas import tpu_sc as plsc

BATCH = 16384
N_FIELDS = 26
NUM_WORKERS = 32
ROWS_PER_W = BATCH // NUM_WORKERS
IDS_PER_W = ROWS_PER_W * N_FIELDS
LANES = 16
CHUNKS = ROWS_PER_W // LANES


def _sc_body(idx_hbm, w_hbm, bias_hbm, out_hbm, idx_v, g_v, bias_v, acc_v, sem):
    wid = lax.axis_index("s") * 2 + lax.axis_index("c")
    base = wid * IDS_PER_W

    pltpu.sync_copy(idx_hbm.at[pl.ds(base, IDS_PER_W)], idx_v)
    pltpu.sync_copy(bias_hbm, bias_v)

    pltpu.async_copy(w_hbm.at[idx_v], g_v, sem).wait()

    bvec = bias_v[...]
    lane_iota = lax.iota(jnp.int32, LANES) * N_FIELDS

    def chunk_body(c, _):
        off = c * (LANES * N_FIELDS)
        acc = bvec
        for f in range(N_FIELDS):
            acc = acc + plsc.load_gather(g_v, [lane_iota + (off + f)])
        acc_v[pl.ds(c * LANES, LANES)] = acc
        return 0

    lax.fori_loop(0, CHUNKS, chunk_body, 0)

    pltpu.sync_copy(acc_v, out_hbm.at[pl.ds(wid * ROWS_PER_W, ROWS_PER_W)])


@jax.jit
def _sc_call(idx_flat, w_flat, bias16):
    mesh = plsc.VectorSubcoreMesh(core_axis_name="c", subcore_axis_name="s")
    fn = pl.kernel(
        _sc_body,
        out_type=jax.ShapeDtypeStruct((BATCH,), jnp.float32),
        mesh=mesh,
        compiler_params=pltpu.CompilerParams(needs_layout_passes=False),
        scratch_types=[
            pltpu.VMEM((IDS_PER_W,), jnp.int32),
            pltpu.VMEM((IDS_PER_W,), jnp.float32),
            pltpu.VMEM((LANES,), jnp.float32),
            pltpu.VMEM((ROWS_PER_W,), jnp.float32),
            pltpu.SemaphoreType.DMA,
        ],
    )
    return fn(idx_flat, w_flat, bias16)


def kernel(indices, weights, bias):
    idx_flat = indices.reshape(-1)
    w_flat = weights.reshape(-1)
    bias16 = jnp.broadcast_to(bias, (LANES,))
    out = _sc_call(idx_flat, w_flat, bias16)
    return out.reshape(BATCH, 1)
